# baseline (speedup 1.0000x reference)
import jax
import jax.numpy as jnp
from jax.experimental import pallas as pl
from jax.experimental.pallas import tpu as pltpu

N_TOK = 4096
D_IN = 1024
N_FEAT = 16384
TOPK = 32


_TM_A = 512
_TF_A = 1024


def _enc_body(x_ref, we_ref, be_ref, h_ref):
    h = jax.lax.dot_general(
        x_ref[...], we_ref[...],
        (((1,), (1,)), ((), ())),
        preferred_element_type=jnp.float32,
        precision=jax.lax.Precision.DEFAULT,
    )
    h_ref[...] = h + be_ref[...][None, :]


def _encoder(x_bf16, W_e_bf16, b_e):
    grid = (N_FEAT // _TF_A, N_TOK // _TM_A)
    return pl.pallas_call(
        _enc_body,
        grid=grid,
        in_specs=[
            pl.BlockSpec((_TM_A, D_IN), lambda f, t: (t, 0)),
            pl.BlockSpec((_TF_A, D_IN), lambda f, t: (f, 0)),
            pl.BlockSpec((_TF_A,), lambda f, t: (f,)),
        ],
        out_specs=pl.BlockSpec((_TM_A, _TF_A), lambda f, t: (t, f)),
        out_shape=jax.ShapeDtypeStruct((N_TOK, N_FEAT), jnp.float32),
    )(x_bf16, W_e_bf16, b_e)


_TM_B = 128
_BISECT_ITERS = 30
_INTERP_ITERS = 10


def _sel_body(h_ref, mask_ref, out_ref, lo_ref, hi_ref, clo_ref, chi_ref,
              done_ref):
    h = h_ref[...]

    cols = [
        jnp.max(h[:, c * 128:(c + 1) * 128], axis=1, keepdims=True)
        for c in range(N_FEAT // 128)
    ]
    cm = jnp.concatenate(cols, axis=1)

    gmax = jnp.max(cm, axis=1, keepdims=True)
    lo0 = jnp.min(cm, axis=1, keepdims=True) - 1.0

    def cm_body(_, carry):
        lo, hi = carry
        m = 0.5 * (lo + hi)
        cnt = jnp.sum((cm > m).astype(jnp.float32), axis=1, keepdims=True)
        pred = cnt >= (TOPK + 1)
        return jnp.where(pred, m, lo), jnp.where(pred, hi, m)

    t0, _ = jax.lax.fori_loop(0, 24, cm_body, (lo0, gmax))

    lo_ref[...] = t0
    hi_ref[...] = gmax
    clo_ref[...] = jnp.zeros_like(gmax) + float(N_FEAT)
    chi_ref[...] = jnp.zeros_like(gmax)
    done_ref[...] = jnp.zeros_like(gmax)

    def body(i, _):
        alldone = jnp.min(done_ref[...]) > 0.5

        @pl.when(jnp.logical_not(alldone))
        def _():
            lo = lo_ref[...]
            hi = hi_ref[...]
            clo = clo_ref[...]
            chi = chi_ref[...]
            done = done_ref[...] > 0.5
            frac = (clo - (TOPK + 0.5)) / jnp.maximum(clo - chi, 1.0)
            frac = jnp.clip(frac, 0.08, 0.92)
            use_interp = jnp.logical_and(i >= 2, i < _INTERP_ITERS)
            f = jnp.where(use_interp, frac, 0.5)
            m = lo + (hi - lo) * f
            cnt = jnp.sum((h > m).astype(jnp.float32), axis=1, keepdims=True)
            pred = cnt >= (TOPK + 1.0)
            live = jnp.logical_not(done)
            move_lo = live & pred
            move_hi = live & jnp.logical_not(pred)
            lo_ref[...] = jnp.where(move_lo, m, lo)
            clo_ref[...] = jnp.where(move_lo, cnt, clo)
            hi_ref[...] = jnp.where(move_hi, m, hi)
            chi_ref[...] = jnp.where(move_hi, cnt, chi)
            done_ref[...] = (done | (cnt == float(TOPK))).astype(jnp.float32)

        return 0

    jax.lax.fori_loop(0, _BISECT_ITERS, body, 0)
    hi = hi_ref[...]
    keep = h > hi
    mask_ref[...] = keep.astype(jnp.float32)
    out_ref[...] = jnp.where(keep & (h > 0.0), h, 0.0)


def _select(h):
    grid = (N_TOK // _TM_B,)
    return pl.pallas_call(
        _sel_body,
        grid=grid,
        in_specs=[pl.BlockSpec((_TM_B, N_FEAT), lambda t: (t, 0))],
        out_specs=[
            pl.BlockSpec((_TM_B, N_FEAT), lambda t: (t, 0)),
            pl.BlockSpec((_TM_B, N_FEAT), lambda t: (t, 0)),
        ],
        out_shape=[
            jax.ShapeDtypeStruct((N_TOK, N_FEAT), jnp.float32),
            jax.ShapeDtypeStruct((N_TOK, N_FEAT), jnp.float32),
        ],
        scratch_shapes=[
            pltpu.VMEM((_TM_B, 1), jnp.float32),
            pltpu.VMEM((_TM_B, 1), jnp.float32),
            pltpu.VMEM((_TM_B, 1), jnp.float32),
            pltpu.VMEM((_TM_B, 1), jnp.float32),
            pltpu.VMEM((_TM_B, 1), jnp.float32),
        ],
    )(h)


_TM_C = 256
_KF_C = 4
_TF_C = N_FEAT // _KF_C


def _dec_body(out_ref, wd_ref, bd_ref, recon_ref, acc_ref):
    f = pl.program_id(0)
    t = pl.program_id(1)
    part = jax.lax.dot_general(
        out_ref[...].astype(jnp.bfloat16), wd_ref[...],
        (((1,), (1,)), ((), ())),
        preferred_element_type=jnp.float32,
    )

    @pl.when(f == 0)
    def _():
        acc_ref[t] = part

    @pl.when(f > 0)
    def _():
        acc_ref[t] += part

    @pl.when(f == _KF_C - 1)
    def _():
        recon_ref[...] = acc_ref[t] + bd_ref[...][None, :]


def _decoder(out, W_d_bf16, b_d):
    grid = (_KF_C, N_TOK // _TM_C)
    return pl.pallas_call(
        _dec_body,
        grid=grid,
        in_specs=[
            pl.BlockSpec((_TM_C, _TF_C), lambda f, t: (t, f)),
            pl.BlockSpec((D_IN, _TF_C), lambda f, t: (0, f)),
            pl.BlockSpec((D_IN,), lambda f, t: (0,)),
        ],
        out_specs=pl.BlockSpec((_TM_C, D_IN), lambda f, t: (t, 0)),
        out_shape=jax.ShapeDtypeStruct((N_TOK, D_IN), jnp.float32),
        scratch_shapes=[
            pltpu.VMEM((N_TOK // _TM_C, _TM_C, D_IN), jnp.float32),
        ],
    )(out, W_d_bf16, b_d)


def kernel(x, W_e, b_e, W_d, b_d):
    h = _encoder(x.astype(jnp.bfloat16), W_e.astype(jnp.bfloat16), b_e)
    mask, out = _select(h)
    recon = _decoder(out, W_d.astype(jnp.bfloat16), b_d)
    return (recon, out, mask)


# speedup vs baseline: 1.2020x; 1.2020x over previous
import jax
import jax.numpy as jnp
from jax.experimental import pallas as pl
from jax.experimental.pallas import tpu as pltpu

N_TOK = 4096
D_IN = 1024
N_FEAT = 16384
TOPK = 32


_TM_A = 512
_TF_A = 1024


def _enc_body(x_ref, we_ref, be_ref, h_ref):
    h = jax.lax.dot_general(
        x_ref[...], we_ref[...],
        (((1,), (1,)), ((), ())),
        preferred_element_type=jnp.float32,
        precision=jax.lax.Precision.DEFAULT,
    )
    h_ref[...] = h + be_ref[...][None, :]


def _encoder(x_bf16, W_e_bf16, b_e):
    grid = (N_FEAT // _TF_A, N_TOK // _TM_A)
    return pl.pallas_call(
        _enc_body,
        grid=grid,
        in_specs=[
            pl.BlockSpec((_TM_A, D_IN), lambda f, t: (t, 0)),
            pl.BlockSpec((_TF_A, D_IN), lambda f, t: (f, 0)),
            pl.BlockSpec((_TF_A,), lambda f, t: (f,)),
        ],
        out_specs=pl.BlockSpec((_TM_A, _TF_A), lambda f, t: (t, f)),
        out_shape=jax.ShapeDtypeStruct((N_TOK, N_FEAT), jnp.float32),
    )(x_bf16, W_e_bf16, b_e)


_TM_B = 128
_BISECT_ITERS = 26


def _sel_body(h_ref, mask_ref, out_ref, lo_ref, hi_ref, done_ref):
    h = h_ref[...]

    cols = [
        jnp.max(h[:, c * 128:(c + 1) * 128], axis=1, keepdims=True)
        for c in range(N_FEAT // 128)
    ]
    cm = jnp.concatenate(cols, axis=1)

    gmax = jnp.max(cm, axis=1, keepdims=True)
    lo0 = jnp.min(cm, axis=1, keepdims=True) - 1.0

    def cm_body(_, carry):
        lo, hi = carry
        m = 0.5 * (lo + hi)
        cnt = jnp.sum((cm > m).astype(jnp.float32), axis=1, keepdims=True)
        pred = cnt >= (TOPK + 1)
        return jnp.where(pred, m, lo), jnp.where(pred, hi, m)

    t0, _ = jax.lax.fori_loop(0, 24, cm_body, (lo0, gmax))

    lo_ref[...] = t0
    hi_ref[...] = gmax
    done_ref[...] = jnp.zeros_like(gmax)

    def body(i, _):
        alldone = jnp.min(done_ref[...]) > 0.5

        @pl.when(jnp.logical_not(alldone))
        def _():
            lo = lo_ref[...]
            hi = hi_ref[...]
            done = done_ref[...] > 0.5
            m = 0.5 * (lo + hi)
            cnt = jnp.sum((h > m).astype(jnp.float32), axis=1, keepdims=True)
            pred = cnt >= (TOPK + 1.0)
            live = jnp.logical_not(done)
            lo_ref[...] = jnp.where(live & pred, m, lo)
            hi_ref[...] = jnp.where(live & jnp.logical_not(pred), m, hi)
            done_ref[...] = (done | (cnt == float(TOPK))).astype(jnp.float32)

        return 0

    jax.lax.fori_loop(0, _BISECT_ITERS, body, 0)
    hi = hi_ref[...]
    keep = h > hi
    mask_ref[...] = keep.astype(jnp.float32)
    out_ref[...] = jnp.where(keep & (h > 0.0), h, 0.0)


def _select(h):
    grid = (N_TOK // _TM_B,)
    return pl.pallas_call(
        _sel_body,
        grid=grid,
        in_specs=[pl.BlockSpec((_TM_B, N_FEAT), lambda t: (t, 0))],
        out_specs=[
            pl.BlockSpec((_TM_B, N_FEAT), lambda t: (t, 0)),
            pl.BlockSpec((_TM_B, N_FEAT), lambda t: (t, 0)),
        ],
        out_shape=[
            jax.ShapeDtypeStruct((N_TOK, N_FEAT), jnp.float32),
            jax.ShapeDtypeStruct((N_TOK, N_FEAT), jnp.float32),
        ],
        scratch_shapes=[
            pltpu.VMEM((_TM_B, 1), jnp.float32),
            pltpu.VMEM((_TM_B, 1), jnp.float32),
            pltpu.VMEM((_TM_B, 1), jnp.float32),
        ],
    )(h)


_TM_C = 256
_KF_C = 4
_TF_C = N_FEAT // _KF_C


def _dec_body(out_ref, wd_ref, bd_ref, recon_ref, acc_ref):
    f = pl.program_id(0)
    t = pl.program_id(1)
    part = jax.lax.dot_general(
        out_ref[...].astype(jnp.bfloat16), wd_ref[...],
        (((1,), (1,)), ((), ())),
        preferred_element_type=jnp.float32,
    )

    @pl.when(f == 0)
    def _():
        acc_ref[t] = part

    @pl.when(f > 0)
    def _():
        acc_ref[t] += part

    @pl.when(f == _KF_C - 1)
    def _():
        recon_ref[...] = acc_ref[t] + bd_ref[...][None, :]


def _decoder(out, W_d_bf16, b_d):
    grid = (_KF_C, N_TOK // _TM_C)
    return pl.pallas_call(
        _dec_body,
        grid=grid,
        in_specs=[
            pl.BlockSpec((_TM_C, _TF_C), lambda f, t: (t, f)),
            pl.BlockSpec((D_IN, _TF_C), lambda f, t: (0, f)),
            pl.BlockSpec((D_IN,), lambda f, t: (0,)),
        ],
        out_specs=pl.BlockSpec((_TM_C, D_IN), lambda f, t: (t, 0)),
        out_shape=jax.ShapeDtypeStruct((N_TOK, D_IN), jnp.float32),
        scratch_shapes=[
            pltpu.VMEM((N_TOK // _TM_C, _TM_C, D_IN), jnp.float32),
        ],
    )(out, W_d_bf16, b_d)


def kernel(x, W_e, b_e, W_d, b_d):
    h = _encoder(x.astype(jnp.bfloat16), W_e.astype(jnp.bfloat16), b_e)
    mask, out = _select(h)
    recon = _decoder(out, W_d.astype(jnp.bfloat16), b_d)
    return (recon, out, mask)
